# Initial kernel scaffold; baseline (speedup 1.0000x reference)
import functools

import jax
import jax.numpy as jnp
from jax import lax
from jax.experimental import pallas as pl
from jax.experimental.pallas import tpu as pltpu
from jax.experimental.pallas import tpu_sc as plsc

N = 10000
F_IN = 128
HID = 64
HALF = 64
E = 320000
NC = 2
NS = 16

CHUNK = 128
NCH = 157
EPT = NCH * CHUNK
EP = NS * EPT
RPT = N // NS
SUB = 125
NSUB = RPT // SUB
DEGP = NS * 640


def _sc_body(x_hbm, src_hbm, dst_hbm, w_hbm, tx_hbm,
             U, A, deg_s, dis_s,
             src_l, dst_l, nrm_l, dis_l, rows, sub, prevb, zeros, buf640):
    c = lax.axis_index("c")
    s = lax.axis_index("s")
    r0 = s * RPT
    d0 = s * 640
    c64 = c * HALF

    z16 = jnp.zeros((16,), jnp.float32)

    def zrow(i, _):
        for g in range(HALF // 16):
            zeros[i, pl.ds(g * 16, 16)] = z16
        return 0
    lax.fori_loop(0, SUB, zrow, 0)

    def z640(i, _):
        buf640[pl.ds(i * 16, 16)] = z16
        return 0
    lax.fori_loop(0, 40, z640, 0)

    pltpu.sync_copy(buf640, deg_s.at[pl.ds(d0, 640)])
    for si in range(NSUB):
        pltpu.sync_copy(zeros, A.at[pl.ds(r0 + si * SUB, SUB)])
    pltpu.sync_copy(x_hbm.at[pl.ds(r0, RPT), pl.ds(c64, HALF)],
                    U.at[pl.ds(r0, RPT)])
    pltpu.sync_copy(src_hbm.at[s], src_l)
    pltpu.sync_copy(dst_hbm.at[s], dst_l)
    pltpu.sync_copy(w_hbm.at[s], nrm_l)
    plsc.subcore_barrier()

    def deg_step(j, _):
        pltpu.sync_copy(nrm_l.at[j], deg_s.at[src_l.at[j]], add=True)
        return 0
    lax.fori_loop(0, NCH, deg_step, 0)
    plsc.subcore_barrier()

    pltpu.sync_copy(deg_s.at[pl.ds(d0, 640)], buf640)

    def dis_step(i, _):
        d = buf640[pl.ds(i * 16, 16)]
        di = plsc.bitcast(d, jnp.int32)
        magic = jnp.full((16,), 0x5F3759DF, jnp.int32)
        y = plsc.bitcast(magic - lax.shift_right_logical(
            di, jnp.full((16,), 1, jnp.int32)), jnp.float32)
        for _ in range(3):
            y = y * (1.5 - 0.5 * d * y * y)
        buf640[pl.ds(i * 16, 16)] = jnp.where(d > 0.0, y, 0.0)
        return 0
    lax.fori_loop(0, 40, dis_step, 0)
    pltpu.sync_copy(buf640, dis_s.at[pl.ds(d0, 640)])
    plsc.subcore_barrier()
    pltpu.sync_copy(dis_s, dis_l)

    def nrm_step(j, _):
        for g in range(CHUNK // 16):
            sl = pl.ds(g * 16, 16)
            a = plsc.load_gather(dis_l, [src_l[j, sl]])
            b = plsc.load_gather(dis_l, [dst_l[j, sl]])
            w = nrm_l[j, sl]
            nrm_l[j, sl] = -(a * w * b)
        return 0
    lax.fori_loop(0, NCH, nrm_step, 0)

    for rep in range(3):
        def prop_step(j, _):
            pltpu.sync_copy(U.at[src_l.at[j]], rows)
            for e in range(CHUNK):
                nb = plsc.load_gather(
                    nrm_l, [jnp.full((16,), j, jnp.int32),
                            jnp.full((16,), e, jnp.int32)])
                for g in range(HALF // 16):
                    sl = pl.ds(g * 16, 16)
                    rows[e, sl] = rows[e, sl] * nb
            pltpu.sync_copy(rows, A.at[dst_l.at[j]], add=True)
            return 0
        lax.fori_loop(0, NCH, prop_step, 0)
        plsc.subcore_barrier()

        for si in range(NSUB):
            row = r0 + si * SUB
            pltpu.sync_copy(A.at[pl.ds(row, SUB)], sub)
            if rep > 0:
                if rep == 1:
                    pltpu.sync_copy(
                        x_hbm.at[pl.ds(row, SUB), pl.ds(c64, HALF)], prevb)
                else:
                    pltpu.sync_copy(tx_hbm.at[0, pl.ds(row, SUB), c], prevb)

                def fix_step(i, _):
                    for g in range(HALF // 16):
                        sl = pl.ds(g * 16, 16)
                        sub[i, sl] = 2.0 * sub[i, sl] - prevb[i, sl]
                    return 0
                lax.fori_loop(0, SUB, fix_step, 0)
            pltpu.sync_copy(sub, tx_hbm.at[rep, pl.ds(row, SUB), c])
            if rep < 2:
                pltpu.sync_copy(sub, U.at[pl.ds(row, SUB)])
                pltpu.sync_copy(zeros, A.at[pl.ds(row, SUB)])
        plsc.subcore_barrier()


_sc_call = functools.partial(
    pl.kernel,
    out_type=jax.ShapeDtypeStruct((3, N, NC, HALF), jnp.float32),
    mesh=plsc.VectorSubcoreMesh(core_axis_name="c", subcore_axis_name="s"),
    scratch_types=[
        pltpu.VMEM_SHARED((N, HALF), jnp.float32),
        pltpu.VMEM_SHARED((N, HALF), jnp.float32),
        pltpu.VMEM_SHARED((DEGP,), jnp.float32),
        pltpu.VMEM_SHARED((DEGP,), jnp.float32),
        pltpu.VMEM((NCH, CHUNK), jnp.int32),
        pltpu.VMEM((NCH, CHUNK), jnp.int32),
        pltpu.VMEM((NCH, CHUNK), jnp.float32),
        pltpu.VMEM((DEGP,), jnp.float32),
        pltpu.VMEM((CHUNK, HALF), jnp.float32),
        pltpu.VMEM((SUB, HALF), jnp.float32),
        pltpu.VMEM((SUB, HALF), jnp.float32),
        pltpu.VMEM((SUB, HALF), jnp.float32),
        pltpu.VMEM((640,), jnp.float32),
    ],
)(_sc_body)


BR = 1000


def _tc_body(x_ref, txa_ref, wcat_ref, brow_ref, wco_ref, wlin_ref,
             blin_ref, out_ref):
    G = jnp.dot(x_ref[...], wcat_ref[0],
                preferred_element_type=jnp.float32)
    for k in range(1, 4):
        G = G + jnp.dot(txa_ref[k - 1], wcat_ref[k],
                        preferred_element_type=jnp.float32)
    G = G + brow_ref[...]
    I = jax.nn.sigmoid(G[:, 0:HID])
    T = jnp.tanh(G[:, HID:2 * HID])
    C = I * T
    O = jax.nn.sigmoid(G[:, 2 * HID:3 * HID] + wco_ref[...] * C)
    h = jax.nn.relu(O * jnp.tanh(C))
    out_ref[...] = (jnp.sum(h * wlin_ref[...], axis=1, keepdims=True)
                    + blin_ref[...])


def kernel(x, edge_index, edge_weight, Wxi, bxi, Whi, bhi, Wxf, bxf, Whf,
           bhf, Wxc, bxc, Whc, bhc, Wxo, bxo, Who, bho, w_ci, w_cf, w_co,
           b_i, b_f, b_c, b_o, W_lin, b_lin):
    src = edge_index[0]
    dst = edge_index[1]
    pad = EP - E
    pad_idx = (jnp.arange(pad, dtype=jnp.int32) * 37) % N
    srcp = jnp.concatenate([src, pad_idx]).reshape(NS, NCH, CHUNK)
    dstp = jnp.concatenate([dst, pad_idx]).reshape(NS, NCH, CHUNK)
    wp = jnp.concatenate(
        [edge_weight, jnp.zeros((pad,), jnp.float32)]).reshape(NS, NCH, CHUNK)

    txa = _sc_call(x, srcp, dstp, wp)
    txa_r = txa.reshape(3, N, F_IN)

    wcat = jnp.stack([
        jnp.concatenate([Wxi[k], Wxc[k], Wxo[k]], axis=1) for k in range(4)
    ])
    brow = jnp.concatenate(
        [bxi + bhi + b_i[0], bxc + bhc + b_c[0],
         bxo + bho + b_o[0]]).reshape(1, 3 * HID)
    wlin = W_lin.reshape(1, HID)
    blin = b_lin.reshape(1, 1)

    return pl.pallas_call(
        _tc_body,
        out_shape=jax.ShapeDtypeStruct((N, 1), jnp.float32),
        grid=(N // BR,),
        in_specs=[
            pl.BlockSpec((BR, F_IN), lambda i: (i, 0)),
            pl.BlockSpec((3, BR, F_IN), lambda i: (0, i, 0)),
            pl.BlockSpec((4, F_IN, 3 * HID), lambda i: (0, 0, 0)),
            pl.BlockSpec((1, 3 * HID), lambda i: (0, 0)),
            pl.BlockSpec((1, HID), lambda i: (0, 0)),
            pl.BlockSpec((1, HID), lambda i: (0, 0)),
            pl.BlockSpec((1, 1), lambda i: (0, 0)),
        ],
        out_specs=pl.BlockSpec((BR, 1), lambda i: (i, 0)),
    )(x, txa_r, wcat, brow, w_co, wlin, blin)


# speedup vs baseline: 6.3631x; 6.3631x over previous
import functools

import jax
import jax.numpy as jnp
from jax import lax
from jax.experimental import pallas as pl
from jax.experimental.pallas import tpu as pltpu
from jax.experimental.pallas import tpu_sc as plsc

N = 10000
F_IN = 128
HID = 64
E = 320000
NC = 2
NS = 16

CHUNK = 128
BLK = 8
NBL = 20
NCH = NBL * BLK
EPT = NCH * CHUNK
EP = NS * EPT
NP = 10240
RPT = NP // NS


def _sc_body(x_hbm, src_hbm, dst_hbm, w_hbm, tx_hbm,
             A, deg_s, dis_s,
             nrm_l, src_b, dst_b, src_i, dst_i, ga, gb, rows, buf640):
    c = lax.axis_index("c")
    s = lax.axis_index("s")
    r0 = s * RPT
    d0 = s * 640

    @pl.when(c == 0)
    def _sc0():
        z16 = jnp.zeros((16,), jnp.float32)

        def zrows(i, _):
            for g in range(F_IN // 16):
                rows[i, pl.ds(g * 16, 16)] = z16
            return 0
        lax.fori_loop(0, CHUNK, zrows, 0)

        def z640(i, _):
            buf640[pl.ds(i * 16, 16)] = z16
            return 0
        lax.fori_loop(0, 40, z640, 0)

        pltpu.sync_copy(buf640, deg_s.at[pl.ds(d0, 640)])
        for si in range(RPT // CHUNK):
            pltpu.sync_copy(rows, A.at[pl.ds(r0 + si * CHUNK, CHUNK)])
        plsc.subcore_barrier()

        def deg_blk(b, _):
            pltpu.sync_copy(src_hbm.at[s, b], src_b)
            pltpu.sync_copy(w_hbm.at[s, b], nrm_l.at[pl.ds(b * BLK, BLK)])

            def deg_chunk(k, _):
                pltpu.sync_copy(nrm_l.at[b * BLK + k],
                                deg_s.at[src_b.at[k]], add=True)
                return 0
            lax.fori_loop(0, BLK, deg_chunk, 0)
            return 0
        lax.fori_loop(0, NBL, deg_blk, 0)
        plsc.subcore_barrier()

        pltpu.sync_copy(deg_s.at[pl.ds(d0, 640)], buf640)

        def dis_step(i, _):
            d = buf640[pl.ds(i * 16, 16)]
            di = lax.bitcast_convert_type(d, jnp.int32)
            magic = jnp.full((16,), 0x5F3759DF, jnp.int32)
            y = lax.bitcast_convert_type(
                magic - lax.shift_right_logical(
                    di, jnp.full((16,), 1, jnp.int32)), jnp.float32)
            for _ in range(3):
                y = y * (1.5 - 0.5 * d * y * y)
            buf640[pl.ds(i * 16, 16)] = jnp.where(d > 0.0, y, 0.0)
            return 0
        lax.fori_loop(0, 40, dis_step, 0)
        pltpu.sync_copy(buf640, dis_s.at[pl.ds(d0, 640)])
        plsc.subcore_barrier()

        def nrm_blk(b, _):
            pltpu.sync_copy(src_hbm.at[s, b], src_b)
            pltpu.sync_copy(dst_hbm.at[s, b], dst_b)

            def nrm_chunk(k, _):
                j = b * BLK + k
                pltpu.sync_copy(dis_s.at[src_b.at[k]], ga)
                pltpu.sync_copy(dis_s.at[dst_b.at[k]], gb)

                def nrm_grp(g, _):
                    sl = pl.ds(g * 16, 16)
                    nrm_l[j, sl] = -(ga[sl] * nrm_l[j, sl] * gb[sl])
                    return 0
                lax.fori_loop(0, CHUNK // 16, nrm_grp, 0)
                return 0
            lax.fori_loop(0, BLK, nrm_chunk, 0)
            return 0
        lax.fori_loop(0, NBL, nrm_blk, 0)

        for rep in range(3):
            src_tab = x_hbm if rep == 0 else tx_hbm.at[rep - 1]

            def prop_blk(b, _):
                pltpu.sync_copy(src_hbm.at[s, b], src_b)
                pltpu.sync_copy(dst_hbm.at[s, b], dst_b)

                def prop_chunk(k, _):
                    j = b * BLK + k

                    def mv(g, _):
                        sl = pl.ds(g * 16, 16)
                        src_i[sl] = src_b[k, sl]
                        dst_i[sl] = dst_b[k, sl]
                        return 0
                    lax.fori_loop(0, CHUNK // 16, mv, 0)
                    pltpu.sync_copy(src_tab.at[src_i], rows)

                    def scale_grp(g, _):
                        n16 = nrm_l[j, pl.ds(g * 16, 16)]
                        r = g * 16
                        for e in range(16):
                            nb = jnp.full((16,), n16[e], jnp.float32)
                            for q in range(F_IN // 16):
                                sl = pl.ds(q * 16, 16)
                                rows[r + e, sl] = rows[r + e, sl] * nb
                        return 0
                    lax.fori_loop(0, CHUNK // 16, scale_grp, 0)
                    pltpu.sync_copy(rows, A.at[dst_i], add=True)
                    return 0
                lax.fori_loop(0, BLK, prop_chunk, 0)
                return 0
            lax.fori_loop(0, NBL, prop_blk, 0)
            plsc.subcore_barrier()

            pltpu.sync_copy(A.at[pl.ds(r0, RPT)],
                            tx_hbm.at[rep, pl.ds(r0, RPT)])
            if rep < 2:
                lax.fori_loop(0, CHUNK, zrows, 0)
                for si in range(RPT // CHUNK):
                    pltpu.sync_copy(rows,
                                    A.at[pl.ds(r0 + si * CHUNK, CHUNK)])
            plsc.subcore_barrier()


@functools.cache
def _make_sc_call():
    return functools.partial(
        pl.kernel,
        out_type=jax.ShapeDtypeStruct((3, NP, F_IN), jnp.float32),
        mesh=plsc.VectorSubcoreMesh(core_axis_name="c", subcore_axis_name="s"),
        scratch_types=[
            pltpu.VMEM_SHARED((NP, F_IN), jnp.float32),
            pltpu.VMEM_SHARED((NP,), jnp.float32),
            pltpu.VMEM_SHARED((NP,), jnp.float32),
            pltpu.VMEM((NCH, CHUNK), jnp.float32),
            pltpu.VMEM((BLK, CHUNK), jnp.int32),
            pltpu.VMEM((BLK, CHUNK), jnp.int32),
            pltpu.VMEM((CHUNK,), jnp.int32),
            pltpu.VMEM((CHUNK,), jnp.int32),
            pltpu.VMEM((CHUNK,), jnp.float32),
            pltpu.VMEM((CHUNK,), jnp.float32),
            pltpu.VMEM((CHUNK, F_IN), jnp.float32),
            pltpu.VMEM((640,), jnp.float32),
        ],
    )(_sc_body)


BR = 1000


def _tc_body(x_ref, txa_ref, wcat_ref, brow_ref, wco_ref,
             wlin_ref, blin_ref, out_ref):
    G = jnp.dot(x_ref[...], wcat_ref[0],
                preferred_element_type=jnp.float32)
    for k in range(3):
        G = G + jnp.dot(txa_ref[k], wcat_ref[k + 1],
                        preferred_element_type=jnp.float32)
    G = G + brow_ref[...]
    I = jax.nn.sigmoid(G[:, 0:HID])
    T = jnp.tanh(G[:, HID:2 * HID])
    C = I * T
    O = jax.nn.sigmoid(G[:, 2 * HID:3 * HID] + wco_ref[...] * C)
    h = jax.nn.relu(O * jnp.tanh(C))
    out_ref[...] = (jnp.sum(h * wlin_ref[...], axis=1, keepdims=True)
                    + blin_ref[...])


def kernel(x, edge_index, edge_weight, Wxi, bxi, Whi, bhi, Wxf, bxf, Whf,
           bhf, Wxc, bxc, Whc, bhc, Wxo, bxo, Who, bho, w_ci, w_cf, w_co,
           b_i, b_f, b_c, b_o, W_lin, b_lin):
    src = edge_index[0]
    dst = edge_index[1]
    pad = EP - E
    pad_idx = (jnp.arange(pad, dtype=jnp.int32) * 37) % N
    srcp = jnp.concatenate([src, pad_idx]).reshape(NS, NBL, BLK, CHUNK)
    dstp = jnp.concatenate([dst, pad_idx]).reshape(NS, NBL, BLK, CHUNK)
    wp = jnp.concatenate(
        [edge_weight, jnp.zeros((pad,), jnp.float32)]
    ).reshape(NS, NBL, BLK, CHUNK)

    x_p = jnp.pad(x, ((0, NP - N), (0, 0)))

    txa = _make_sc_call()(x_p, srcp, dstp, wp)

    wcat = jnp.stack([
        jnp.concatenate([Wxi[k], Wxc[k], Wxo[k]], axis=1) for k in range(4)
    ])
    wcat = jnp.stack([wcat[0] - wcat[2], wcat[1] - 3.0 * wcat[3],
                      2.0 * wcat[2], 4.0 * wcat[3]])

    brow = jnp.concatenate(
        [bxi + bhi + b_i[0], bxc + bhc + b_c[0],
         bxo + bho + b_o[0]]).reshape(1, 3 * HID)
    wlin = W_lin.reshape(1, HID)
    blin = b_lin.reshape(1, 1)

    return pl.pallas_call(
        _tc_body,
        out_shape=jax.ShapeDtypeStruct((N, 1), jnp.float32),
        grid=(N // BR,),
        in_specs=[
            pl.BlockSpec((BR, F_IN), lambda i: (i, 0)),
            pl.BlockSpec((3, BR, F_IN), lambda i: (0, i, 0)),
            pl.BlockSpec((4, F_IN, 3 * HID), lambda i: (0, 0, 0)),
            pl.BlockSpec((1, 3 * HID), lambda i: (0, 0)),
            pl.BlockSpec((1, HID), lambda i: (0, 0)),
            pl.BlockSpec((1, HID), lambda i: (0, 0)),
            pl.BlockSpec((1, 1), lambda i: (0, 0)),
        ],
        out_specs=pl.BlockSpec((BR, 1), lambda i: (i, 0)),
    )(x, txa, wcat, brow, w_co, wlin, blin)


# speedup vs baseline: 8.7563x; 1.3761x over previous
import functools

import jax
import jax.numpy as jnp
from jax import lax
from jax.experimental import pallas as pl
from jax.experimental.pallas import tpu as pltpu
from jax.experimental.pallas import tpu_sc as plsc

N = 10000
F_IN = 128
HID = 64
E = 320000
NC = 2
NS = 16

CHUNK = 128
BLK = 8
NBL = 20
NCH = NBL * BLK
EPT = NCH * CHUNK
EP = NS * EPT
NP = 10240
RPT = NP // NS


def _sc_body(x_hbm, src_hbm, dst_hbm, w_hbm, tx_hbm,
             A, deg_s, dis_s, nrm_hbm,
             nrm_b, src_b, dst_b, src_i, dst_i, src_i2, dst_i2,
             ga, gb, rows, rows2, buf640, gs0, gs1, ss0, ss1):
    c = lax.axis_index("c")
    s = lax.axis_index("s")
    r0 = s * RPT
    d0 = s * 640

    @pl.when(c == 0)
    def _sc0():
        z16 = jnp.zeros((16,), jnp.float32)

        def zrows(i, _):
            for g in range(F_IN // 16):
                rows[i, pl.ds(g * 16, 16)] = z16
            return 0
        lax.fori_loop(0, CHUNK, zrows, 0)

        def z640(i, _):
            buf640[pl.ds(i * 16, 16)] = z16
            return 0
        lax.fori_loop(0, 40, z640, 0)

        pltpu.sync_copy(buf640, deg_s.at[pl.ds(d0, 640)])
        for si in range(RPT // CHUNK):
            pltpu.sync_copy(rows, A.at[pl.ds(r0 + si * CHUNK, CHUNK)])
        plsc.subcore_barrier()

        def deg_blk(b, _):
            pltpu.sync_copy(src_hbm.at[s, b], src_b)
            pltpu.sync_copy(w_hbm.at[s, b], nrm_b)

            def deg_chunk(k, _):
                pltpu.sync_copy(nrm_b.at[k], deg_s.at[src_b.at[k]], add=True)
                return 0
            lax.fori_loop(0, BLK, deg_chunk, 0)
            return 0
        lax.fori_loop(0, NBL, deg_blk, 0)
        plsc.subcore_barrier()

        pltpu.sync_copy(deg_s.at[pl.ds(d0, 640)], buf640)

        def dis_step(i, _):
            d = buf640[pl.ds(i * 16, 16)]
            di = lax.bitcast_convert_type(d, jnp.int32)
            magic = jnp.full((16,), 0x5F3759DF, jnp.int32)
            y = lax.bitcast_convert_type(
                magic - lax.shift_right_logical(
                    di, jnp.full((16,), 1, jnp.int32)), jnp.float32)
            for _ in range(3):
                y = y * (1.5 - 0.5 * d * y * y)
            buf640[pl.ds(i * 16, 16)] = jnp.where(d > 0.0, y, 0.0)
            return 0
        lax.fori_loop(0, 40, dis_step, 0)
        pltpu.sync_copy(buf640, dis_s.at[pl.ds(d0, 640)])
        plsc.subcore_barrier()

        def nrm_blk(b, _):
            pltpu.sync_copy(src_hbm.at[s, b], src_b)
            pltpu.sync_copy(dst_hbm.at[s, b], dst_b)
            pltpu.sync_copy(w_hbm.at[s, b], nrm_b)

            def nrm_chunk(k, _):
                pltpu.sync_copy(dis_s.at[src_b.at[k]], ga)
                pltpu.sync_copy(dis_s.at[dst_b.at[k]], gb)

                def nrm_grp(g, _):
                    sl = pl.ds(g * 16, 16)
                    nrm_b[k, sl] = -(ga[sl] * nrm_b[k, sl] * gb[sl])
                    return 0
                lax.fori_loop(0, CHUNK // 16, nrm_grp, 0)
                return 0
            lax.fori_loop(0, BLK, nrm_chunk, 0)
            pltpu.sync_copy(nrm_b, nrm_hbm.at[s, b])
            return 0
        lax.fori_loop(0, NBL, nrm_blk, 0)

        rowsb = (rows, rows2)
        srcib = (src_i, src_i2)
        dstib = (dst_i, dst_i2)
        gsems = (gs0, gs1)
        ssems = (ss0, ss1)

        def mv_idx(k, sip, dip):
            def mvg(g, _):
                sl = pl.ds(g * 16, 16)
                sip[sl] = src_b[k, sl]
                dip[sl] = dst_b[k, sl]
                return 0
            lax.fori_loop(0, CHUNK // 16, mvg, 0)

        for rep in range(3):
            src_tab = x_hbm if rep == 0 else tx_hbm.at[rep - 1]

            def prop_blk(b, _):
                pltpu.sync_copy(src_hbm.at[s, b], src_b)
                pltpu.sync_copy(dst_hbm.at[s, b], dst_b)
                pltpu.sync_copy(nrm_hbm.at[s, b], nrm_b)
                mv_idx(0, srcib[0], dstib[0])
                pltpu.async_copy(src_tab.at[srcib[0]], rowsb[0], gsems[0])

                def pair(kk, _):
                    for ph in range(2):
                        k = 2 * kk + ph
                        p, q = ph, 1 - ph

                        @pl.when(k >= 1)
                        def _():
                            pltpu.make_async_copy(
                                rowsb[q], A.at[dstib[q]], ssems[q]).wait()

                        @pl.when(k + 1 < BLK)
                        def _():
                            mv_idx(k + 1, srcib[q], dstib[q])
                            pltpu.async_copy(
                                src_tab.at[srcib[q]], rowsb[q], gsems[q])

                        pltpu.make_async_copy(
                            src_tab.at[srcib[p]], rowsb[p], gsems[p]).wait()

                        def scale_grp(g, _):
                            n16 = nrm_b[k, pl.ds(g * 16, 16)]
                            r = g * 16
                            for e in range(16):
                                nb = jnp.full((16,), n16[e], jnp.float32)
                                for qq in range(F_IN // 16):
                                    sl = pl.ds(qq * 16, 16)
                                    rowsb[p][r + e, sl] = (
                                        rowsb[p][r + e, sl] * nb)
                            return 0
                        lax.fori_loop(0, CHUNK // 16, scale_grp, 0)
                        pltpu.async_copy(
                            rowsb[p], A.at[dstib[p]], ssems[p], add=True)
                    return 0
                lax.fori_loop(0, BLK // 2, pair, 0)
                pltpu.make_async_copy(
                    rowsb[1], A.at[dstib[1]], ssems[1]).wait()
                return 0
            lax.fori_loop(0, NBL, prop_blk, 0)
            plsc.subcore_barrier()

            pltpu.sync_copy(A.at[pl.ds(r0, RPT)],
                            tx_hbm.at[rep, pl.ds(r0, RPT)])
            if rep < 2:
                lax.fori_loop(0, CHUNK, zrows, 0)
                for si in range(RPT // CHUNK):
                    pltpu.sync_copy(rows,
                                    A.at[pl.ds(r0 + si * CHUNK, CHUNK)])
            plsc.subcore_barrier()


@functools.cache
def _make_sc_call():
    return functools.partial(
        pl.kernel,
        out_type=jax.ShapeDtypeStruct((3, NP, F_IN), jnp.float32),
        mesh=plsc.VectorSubcoreMesh(core_axis_name="c", subcore_axis_name="s"),
        scratch_types=[
            pltpu.VMEM_SHARED((NP, F_IN), jnp.float32),
            pltpu.VMEM_SHARED((NP,), jnp.float32),
            pltpu.VMEM_SHARED((NP,), jnp.float32),
            pltpu.HBM((NS, NBL, BLK, CHUNK), jnp.float32),
            pltpu.VMEM((BLK, CHUNK), jnp.float32),
            pltpu.VMEM((BLK, CHUNK), jnp.int32),
            pltpu.VMEM((BLK, CHUNK), jnp.int32),
            pltpu.VMEM((CHUNK,), jnp.int32),
            pltpu.VMEM((CHUNK,), jnp.int32),
            pltpu.VMEM((CHUNK,), jnp.int32),
            pltpu.VMEM((CHUNK,), jnp.int32),
            pltpu.VMEM((CHUNK,), jnp.float32),
            pltpu.VMEM((CHUNK,), jnp.float32),
            pltpu.VMEM((CHUNK, F_IN), jnp.float32),
            pltpu.VMEM((CHUNK, F_IN), jnp.float32),
            pltpu.VMEM((640,), jnp.float32),
            pltpu.SemaphoreType.DMA,
            pltpu.SemaphoreType.DMA,
            pltpu.SemaphoreType.DMA,
            pltpu.SemaphoreType.DMA,
        ],
    )(_sc_body)


BR = 1000


def _tc_body(x_ref, txa_ref, wcat_ref, brow_ref, wco_ref,
             wlin_ref, blin_ref, out_ref):
    G = jnp.dot(x_ref[...], wcat_ref[0],
                preferred_element_type=jnp.float32)
    for k in range(3):
        G = G + jnp.dot(txa_ref[k], wcat_ref[k + 1],
                        preferred_element_type=jnp.float32)
    G = G + brow_ref[...]
    I = jax.nn.sigmoid(G[:, 0:HID])
    T = jnp.tanh(G[:, HID:2 * HID])
    C = I * T
    O = jax.nn.sigmoid(G[:, 2 * HID:3 * HID] + wco_ref[...] * C)
    h = jax.nn.relu(O * jnp.tanh(C))
    out_ref[...] = (jnp.sum(h * wlin_ref[...], axis=1, keepdims=True)
                    + blin_ref[...])


def kernel(x, edge_index, edge_weight, Wxi, bxi, Whi, bhi, Wxf, bxf, Whf,
           bhf, Wxc, bxc, Whc, bhc, Wxo, bxo, Who, bho, w_ci, w_cf, w_co,
           b_i, b_f, b_c, b_o, W_lin, b_lin):
    src = edge_index[0]
    dst = edge_index[1]
    pad = EP - E
    pad_idx = (jnp.arange(pad, dtype=jnp.int32) * 37) % N
    srcp = jnp.concatenate([src, pad_idx]).reshape(NS, NBL, BLK, CHUNK)
    dstp = jnp.concatenate([dst, pad_idx]).reshape(NS, NBL, BLK, CHUNK)
    wp = jnp.concatenate(
        [edge_weight, jnp.zeros((pad,), jnp.float32)]
    ).reshape(NS, NBL, BLK, CHUNK)

    x_p = jnp.pad(x, ((0, NP - N), (0, 0)))

    txa = _make_sc_call()(x_p, srcp, dstp, wp)

    wcat = jnp.stack([
        jnp.concatenate([Wxi[k], Wxc[k], Wxo[k]], axis=1) for k in range(4)
    ])
    wcat = jnp.stack([wcat[0] - wcat[2], wcat[1] - 3.0 * wcat[3],
                      2.0 * wcat[2], 4.0 * wcat[3]])

    brow = jnp.concatenate(
        [bxi + bhi + b_i[0], bxc + bhc + b_c[0],
         bxo + bho + b_o[0]]).reshape(1, 3 * HID)
    wlin = W_lin.reshape(1, HID)
    blin = b_lin.reshape(1, 1)

    return pl.pallas_call(
        _tc_body,
        out_shape=jax.ShapeDtypeStruct((N, 1), jnp.float32),
        grid=(N // BR,),
        in_specs=[
            pl.BlockSpec((BR, F_IN), lambda i: (i, 0)),
            pl.BlockSpec((3, BR, F_IN), lambda i: (0, i, 0)),
            pl.BlockSpec((4, F_IN, 3 * HID), lambda i: (0, 0, 0)),
            pl.BlockSpec((1, 3 * HID), lambda i: (0, 0)),
            pl.BlockSpec((1, HID), lambda i: (0, 0)),
            pl.BlockSpec((1, HID), lambda i: (0, 0)),
            pl.BlockSpec((1, 1), lambda i: (0, 0)),
        ],
        out_specs=pl.BlockSpec((BR, 1), lambda i: (i, 0)),
    )(x, txa, wcat, brow, w_co, wlin, blin)


# speedup vs baseline: 10.0221x; 1.1446x over previous
import functools

import jax
import jax.numpy as jnp
from jax import lax
from jax.experimental import pallas as pl
from jax.experimental.pallas import tpu as pltpu
from jax.experimental.pallas import tpu_sc as plsc

N = 10000
F_IN = 128
HID = 64
E = 320000
NC = 2
NS = 16

CHUNK = 128
BLK = 8
NBL = 20
NCH = NBL * BLK
EPT = NCH * CHUNK
EP = NS * EPT
NP = 10240
RPT = NP // NS


def _sc_body(x_hbm, src_hbm, dst_hbm, w_hbm, tx_hbm,
             A, deg_s, dis_s, nrm_hbm,
             nrm_b, src_b, dst_b, src_i, dst_i, src_i2, dst_i2,
             ga, gb, rows, rows2, buf640, gs0, gs1, ss0, ss1):
    c = lax.axis_index("c")
    s = lax.axis_index("s")
    r0 = s * RPT
    d0 = s * 640

    @pl.when(c == 0)
    def _sc0():
        z16 = jnp.zeros((16,), jnp.float32)

        def zrows(i, _):
            for g in range(F_IN // 16):
                rows[i, pl.ds(g * 16, 16)] = z16
            return 0
        lax.fori_loop(0, CHUNK, zrows, 0)

        def z640(i, _):
            buf640[pl.ds(i * 16, 16)] = z16
            return 0
        lax.fori_loop(0, 40, z640, 0)

        pltpu.sync_copy(buf640, deg_s.at[pl.ds(d0, 640)])
        for si in range(RPT // CHUNK):
            pltpu.sync_copy(rows, A.at[pl.ds(r0 + si * CHUNK, CHUNK)])
        plsc.subcore_barrier()

        def deg_blk(b, _):
            pltpu.sync_copy(src_hbm.at[s, b], src_b)
            pltpu.sync_copy(w_hbm.at[s, b], nrm_b)

            def deg_fire(k, _):
                pltpu.async_copy(nrm_b.at[k], deg_s.at[src_b.at[k]], gs0,
                                 add=True)
                return 0
            lax.fori_loop(0, BLK, deg_fire, 0)

            def deg_drain(k, _):
                pltpu.make_async_copy(
                    nrm_b.at[k], deg_s.at[src_b.at[k]], gs0).wait()
                return 0
            lax.fori_loop(0, BLK, deg_drain, 0)
            return 0
        lax.fori_loop(0, NBL, deg_blk, 0)
        plsc.subcore_barrier()

        pltpu.sync_copy(deg_s.at[pl.ds(d0, 640)], buf640)

        def dis_step(i, _):
            d = buf640[pl.ds(i * 16, 16)]
            di = lax.bitcast_convert_type(d, jnp.int32)
            magic = jnp.full((16,), 0x5F3759DF, jnp.int32)
            y = lax.bitcast_convert_type(
                magic - lax.shift_right_logical(
                    di, jnp.full((16,), 1, jnp.int32)), jnp.float32)
            for _ in range(3):
                y = y * (1.5 - 0.5 * d * y * y)
            buf640[pl.ds(i * 16, 16)] = jnp.where(d > 0.0, y, 0.0)
            return 0
        lax.fori_loop(0, 40, dis_step, 0)
        pltpu.sync_copy(buf640, dis_s.at[pl.ds(d0, 640)])
        plsc.subcore_barrier()

        def nrm_blk(b, _):
            pltpu.sync_copy(src_hbm.at[s, b], src_b)
            pltpu.sync_copy(dst_hbm.at[s, b], dst_b)
            pltpu.sync_copy(w_hbm.at[s, b], nrm_b)

            def nrm_fire(k, _):
                pltpu.async_copy(dis_s.at[src_b.at[k]], ga.at[k], gs0)
                pltpu.async_copy(dis_s.at[dst_b.at[k]], gb.at[k], gs1)
                return 0
            lax.fori_loop(0, BLK, nrm_fire, 0)

            def nrm_chunk(k, _):
                pltpu.make_async_copy(
                    dis_s.at[src_b.at[k]], ga.at[k], gs0).wait()
                pltpu.make_async_copy(
                    dis_s.at[dst_b.at[k]], gb.at[k], gs1).wait()

                def nrm_grp(g, _):
                    sl = pl.ds(g * 16, 16)
                    nrm_b[k, sl] = -(ga[k, sl] * nrm_b[k, sl] * gb[k, sl])
                    return 0
                lax.fori_loop(0, CHUNK // 16, nrm_grp, 0)
                return 0
            lax.fori_loop(0, BLK, nrm_chunk, 0)
            pltpu.sync_copy(nrm_b, nrm_hbm.at[s, b])
            return 0
        lax.fori_loop(0, NBL, nrm_blk, 0)

        rowsb = (rows, rows2)
        srcib = (src_i, src_i2)
        dstib = (dst_i, dst_i2)
        gsems = (gs0, gs1)
        ssems = (ss0, ss1)

        def mv_idx(k, sip, dip):
            def mvg(g, _):
                sl = pl.ds(g * 16, 16)
                sip[sl] = src_b[k, sl]
                dip[sl] = dst_b[k, sl]
                return 0
            lax.fori_loop(0, CHUNK // 16, mvg, 0)

        for rep in range(3):
            src_tab = x_hbm if rep == 0 else tx_hbm.at[rep - 1]

            def prop_blk(b, _):
                pltpu.sync_copy(src_hbm.at[s, b], src_b)
                pltpu.sync_copy(dst_hbm.at[s, b], dst_b)
                pltpu.sync_copy(nrm_hbm.at[s, b], nrm_b)
                mv_idx(0, srcib[0], dstib[0])
                pltpu.async_copy(src_tab.at[srcib[0]], rowsb[0], gsems[0])

                def pair(kk, _):
                    for ph in range(2):
                        k = 2 * kk + ph
                        p, q = ph, 1 - ph

                        @pl.when(k >= 1)
                        def _():
                            pltpu.make_async_copy(
                                rowsb[q], A.at[dstib[q]], ssems[q]).wait()

                        @pl.when(k + 1 < BLK)
                        def _():
                            mv_idx(k + 1, srcib[q], dstib[q])
                            pltpu.async_copy(
                                src_tab.at[srcib[q]], rowsb[q], gsems[q])

                        pltpu.make_async_copy(
                            src_tab.at[srcib[p]], rowsb[p], gsems[p]).wait()

                        def scale_grp(g, _):
                            n16 = nrm_b[k, pl.ds(g * 16, 16)]
                            r = g * 16
                            for e in range(16):
                                nb = jnp.full((16,), n16[e], jnp.float32)
                                for qq in range(F_IN // 16):
                                    sl = pl.ds(qq * 16, 16)
                                    rowsb[p][r + e, sl] = (
                                        rowsb[p][r + e, sl] * nb)
                            return 0
                        lax.fori_loop(0, CHUNK // 16, scale_grp, 0)
                        pltpu.async_copy(
                            rowsb[p], A.at[dstib[p]], ssems[p], add=True)
                    return 0
                lax.fori_loop(0, BLK // 2, pair, 0)
                pltpu.make_async_copy(
                    rowsb[1], A.at[dstib[1]], ssems[1]).wait()
                return 0
            lax.fori_loop(0, NBL, prop_blk, 0)
            plsc.subcore_barrier()

            pltpu.sync_copy(A.at[pl.ds(r0, RPT)],
                            tx_hbm.at[rep, pl.ds(r0, RPT)])
            if rep < 2:
                lax.fori_loop(0, CHUNK, zrows, 0)
                for si in range(RPT // CHUNK):
                    pltpu.sync_copy(rows,
                                    A.at[pl.ds(r0 + si * CHUNK, CHUNK)])
            plsc.subcore_barrier()


@functools.cache
def _make_sc_call():
    return functools.partial(
        pl.kernel,
        out_type=jax.ShapeDtypeStruct((3, NP, F_IN), jnp.float32),
        mesh=plsc.VectorSubcoreMesh(core_axis_name="c", subcore_axis_name="s"),
        scratch_types=[
            pltpu.VMEM_SHARED((NP, F_IN), jnp.float32),
            pltpu.VMEM_SHARED((NP,), jnp.float32),
            pltpu.VMEM_SHARED((NP,), jnp.float32),
            pltpu.HBM((NS, NBL, BLK, CHUNK), jnp.float32),
            pltpu.VMEM((BLK, CHUNK), jnp.float32),
            pltpu.VMEM((BLK, CHUNK), jnp.int32),
            pltpu.VMEM((BLK, CHUNK), jnp.int32),
            pltpu.VMEM((CHUNK,), jnp.int32),
            pltpu.VMEM((CHUNK,), jnp.int32),
            pltpu.VMEM((CHUNK,), jnp.int32),
            pltpu.VMEM((CHUNK,), jnp.int32),
            pltpu.VMEM((BLK, CHUNK), jnp.float32),
            pltpu.VMEM((BLK, CHUNK), jnp.float32),
            pltpu.VMEM((CHUNK, F_IN), jnp.float32),
            pltpu.VMEM((CHUNK, F_IN), jnp.float32),
            pltpu.VMEM((640,), jnp.float32),
            pltpu.SemaphoreType.DMA,
            pltpu.SemaphoreType.DMA,
            pltpu.SemaphoreType.DMA,
            pltpu.SemaphoreType.DMA,
        ],
    )(_sc_body)


BR = 1000


def _tc_body(x_ref, txa_ref, wcat_ref, brow_ref, wco_ref,
             wlin_ref, blin_ref, out_ref):
    G = jnp.dot(x_ref[...], wcat_ref[0],
                preferred_element_type=jnp.float32)
    for k in range(3):
        G = G + jnp.dot(txa_ref[k], wcat_ref[k + 1],
                        preferred_element_type=jnp.float32)
    G = G + brow_ref[...]
    I = jax.nn.sigmoid(G[:, 0:HID])
    T = jnp.tanh(G[:, HID:2 * HID])
    C = I * T
    O = jax.nn.sigmoid(G[:, 2 * HID:3 * HID] + wco_ref[...] * C)
    h = jax.nn.relu(O * jnp.tanh(C))
    out_ref[...] = (jnp.sum(h * wlin_ref[...], axis=1, keepdims=True)
                    + blin_ref[...])


def kernel(x, edge_index, edge_weight, Wxi, bxi, Whi, bhi, Wxf, bxf, Whf,
           bhf, Wxc, bxc, Whc, bhc, Wxo, bxo, Who, bho, w_ci, w_cf, w_co,
           b_i, b_f, b_c, b_o, W_lin, b_lin):
    src = edge_index[0]
    dst = edge_index[1]
    pad = EP - E
    pad_idx = (jnp.arange(pad, dtype=jnp.int32) * 37) % N
    srcp = jnp.concatenate([src, pad_idx]).reshape(NS, NBL, BLK, CHUNK)
    dstp = jnp.concatenate([dst, pad_idx]).reshape(NS, NBL, BLK, CHUNK)
    wp = jnp.concatenate(
        [edge_weight, jnp.zeros((pad,), jnp.float32)]
    ).reshape(NS, NBL, BLK, CHUNK)

    x_p = jnp.pad(x, ((0, NP - N), (0, 0)))

    txa = _make_sc_call()(x_p, srcp, dstp, wp)

    wcat = jnp.stack([
        jnp.concatenate([Wxi[k], Wxc[k], Wxo[k]], axis=1) for k in range(4)
    ])
    wcat = jnp.stack([wcat[0] - wcat[2], wcat[1] - 3.0 * wcat[3],
                      2.0 * wcat[2], 4.0 * wcat[3]])

    brow = jnp.concatenate(
        [bxi + bhi + b_i[0], bxc + bhc + b_c[0],
         bxo + bho + b_o[0]]).reshape(1, 3 * HID)
    wlin = W_lin.reshape(1, HID)
    blin = b_lin.reshape(1, 1)

    return pl.pallas_call(
        _tc_body,
        out_shape=jax.ShapeDtypeStruct((N, 1), jnp.float32),
        grid=(N // BR,),
        in_specs=[
            pl.BlockSpec((BR, F_IN), lambda i: (i, 0)),
            pl.BlockSpec((3, BR, F_IN), lambda i: (0, i, 0)),
            pl.BlockSpec((4, F_IN, 3 * HID), lambda i: (0, 0, 0)),
            pl.BlockSpec((1, 3 * HID), lambda i: (0, 0)),
            pl.BlockSpec((1, HID), lambda i: (0, 0)),
            pl.BlockSpec((1, HID), lambda i: (0, 0)),
            pl.BlockSpec((1, 1), lambda i: (0, 0)),
        ],
        out_specs=pl.BlockSpec((BR, 1), lambda i: (i, 0)),
    )(x, txa, wcat, brow, w_co, wlin, blin)
